# baseline (speedup 1.0000x reference)
import functools

import jax
import jax.numpy as jnp
from jax import lax
from jax.experimental import pallas as pl
from jax.experimental.pallas import tpu as pltpu
from jax.experimental.pallas import tpu_sc as plsc

_EMB = 64
_PATH = 20
_ROWS = 32
_L = 16


def _log_f32(x):
    bits = lax.bitcast_convert_type(x, jnp.int32)
    e = (bits >> 23) - 127
    mbits = (bits & 0x007FFFFF) | 0x3F800000
    m = lax.bitcast_convert_type(mbits, jnp.float32)
    big = m > 1.4142135381698608
    m = jnp.where(big, m * 0.5, m)
    ef = e.astype(jnp.float32) + jnp.where(big, 1.0, 0.0)
    t = (m - 1.0) / (m + 1.0)
    t2 = t * t
    poly = 1.0 + t2 * (
        0.3333333432674408
        + t2 * (0.20000000298023224 + t2 * (0.1428571492433548 + t2 * 0.1111111119389534))
    )
    return ef * 0.6931471805599453 + 2.0 * t * poly


def _sc_body(in_table, node_table, idx_hbm, tgt_hbm, codes_hbm, out_hbm,
             idx_v, tgt_v, codes_v, u_rows, v_rows, out_v, sem_u, sem_v):
    cid = lax.axis_index("c")
    sid = lax.axis_index("s")

    @pl.when(jnp.logical_and(cid == 0, sid == 0))
    def _():
        pltpu.sync_copy(idx_hbm, idx_v)
        pltpu.sync_copy(tgt_hbm, tgt_v)
        pltpu.sync_copy(codes_hbm, codes_v)

        cp_u = pltpu.make_async_copy(node_table.at[idx_v], u_rows, sem_u)
        cp_v = pltpu.make_async_copy(in_table.at[tgt_v], v_rows, sem_v)
        cp_u.start()
        cp_v.start()
        cp_u.wait()
        cp_v.wait()

        vch = [v_rows[0, pl.ds(j * _L, _L)] for j in range(_EMB // _L)]

        lane = lax.iota(jnp.int32, 16)
        acc0 = jnp.zeros((_L,), jnp.float32)
        acc1 = jnp.zeros((_L,), jnp.float32)
        for i in range(_PATH):
            w = u_rows[i, pl.ds(0, _L)] * vch[0]
            for j in range(1, _EMB // _L):
                w = w + u_rows[i, pl.ds(j * _L, _L)] * vch[j]
            sv = jnp.broadcast_to(jnp.sum(w), (_L,))
            if i < _L:
                acc0 = jnp.where(lane == i, sv, acc0)
            else:
                acc1 = jnp.where(lane == (i - _L), sv, acc1)

        c0 = codes_v[pl.ds(0, _L)]
        c1 = codes_v[pl.ds(_L, _L)]
        z0 = jnp.where(c0 == 1, acc0, -acc0)
        z1 = jnp.where(c1 == 1, acc1, -acc1)

        p0 = 1.0 / (1.0 + jnp.exp(-z0))
        p1 = 1.0 / (1.0 + jnp.exp(-z1))
        l0 = _log_f32(p0 + 1e-9)
        l1 = _log_f32(p1 + 1e-9)

        lane = lax.iota(jnp.int32, 16)
        l1 = jnp.where(lane < (_PATH - _L), l1, 0.0)
        loss = -jnp.sum(l0 + l1)

        out_v[...] = jnp.broadcast_to(loss, (_L,))
        pltpu.sync_copy(out_v, out_hbm)


@functools.cache
def _build_sc_fn():
  return pl.kernel(
    _sc_body,
    out_type=jax.ShapeDtypeStruct((_L,), jnp.float32),
    mesh=plsc.VectorSubcoreMesh(core_axis_name="c", subcore_axis_name="s"),
    scratch_types=[
        pltpu.VMEM((_ROWS,), jnp.int32),
        pltpu.VMEM((8,), jnp.int32),
        pltpu.VMEM((_ROWS,), jnp.int32),
        pltpu.VMEM((_ROWS, _EMB), jnp.float32),
        pltpu.VMEM((8, _EMB), jnp.float32),
        pltpu.VMEM((_L,), jnp.float32),
        pltpu.SemaphoreType.DMA,
        pltpu.SemaphoreType.DMA,
    ],
    compiler_params=pltpu.CompilerParams(
        needs_layout_passes=False, use_tc_tiling_on_sc=False
    ),
  )


@jax.jit
def kernel(in_table, node_table, target_idx, node_ids, codes):
    idx_pad = jnp.zeros((_ROWS,), jnp.int32).at[:_PATH].set(node_ids.astype(jnp.int32))
    tgt_pad = jnp.zeros((8,), jnp.int32).at[0].set(target_idx.astype(jnp.int32))
    codes_pad = jnp.zeros((_ROWS,), jnp.int32).at[:_PATH].set(codes.astype(jnp.int32))
    out = _build_sc_fn()(in_table, node_table, idx_pad, tgt_pad, codes_pad)
    return out[0]


# speedup vs baseline: 1.5942x; 1.5942x over previous
import functools

import jax
import jax.numpy as jnp
from jax import lax
from jax.experimental import pallas as pl
from jax.experimental.pallas import tpu as pltpu
from jax.experimental.pallas import tpu_sc as plsc

_EMB = 64
_PATH = 20
_ROWS = 32
_L = 16


def _log_f32(x):
    bits = lax.bitcast_convert_type(x, jnp.int32)
    e = (bits >> 23) - 127
    mbits = (bits & 0x007FFFFF) | 0x3F800000
    m = lax.bitcast_convert_type(mbits, jnp.float32)
    big = m > 1.4142135381698608
    m = jnp.where(big, m * 0.5, m)
    ef = e.astype(jnp.float32) + jnp.where(big, 1.0, 0.0)
    t = (m - 1.0) / (m + 1.0)
    t2 = t * t
    poly = 1.0 + t2 * (
        0.3333333432674408
        + t2 * (0.20000000298023224 + t2 * (0.1428571492433548 + t2 * 0.1111111119389534))
    )
    return ef * 0.6931471805599453 + 2.0 * t * poly


def _sc_body(in_table, node_table, idx_hbm, tgt_hbm, codes_hbm, out_hbm,
             idx_vtmp, codes_v, u_rows, v_rows, out_v, sem_u, sem_v):
    cid = lax.axis_index("c")
    sid = lax.axis_index("s")

    @pl.when(jnp.logical_and(cid == 0, sid == 0))
    def _():
        pltpu.sync_copy(idx_hbm, idx_vtmp.at[pl.ds(0, _ROWS)])
        pltpu.sync_copy(tgt_hbm, idx_vtmp.at[pl.ds(_ROWS, 8)])
        pltpu.sync_copy(codes_hbm, codes_v)

        lane0 = lax.iota(jnp.int32, 16)
        iv0 = idx_vtmp[pl.ds(0, _L)]
        iv1 = idx_vtmp[pl.ds(_L, _L)]
        idx_scalars = [
            jnp.sum(jnp.where(lane0 == i, iv0, 0)) for i in range(_L)
        ] + [
            jnp.sum(jnp.where(lane0 == (i - _L), iv1, 0)) for i in range(_L, _PATH)
        ]
        tgt_scalar = jnp.sum(jnp.where(lane0 == 0, idx_vtmp[pl.ds(_ROWS, _L)], 0))

        cps = []
        for i in range(_PATH):
            cps.append(pltpu.make_async_copy(
                node_table.at[idx_scalars[i]], u_rows.at[i], sem_u))
        cps.append(pltpu.make_async_copy(
            in_table.at[tgt_scalar], v_rows.at[0], sem_v))
        for cp in cps:
            cp.start()
        for cp in cps:
            cp.wait()

        vch = [v_rows[0, pl.ds(j * _L, _L)] for j in range(_EMB // _L)]

        lane = lax.iota(jnp.int32, 16)
        acc0 = jnp.zeros((_L,), jnp.float32)
        acc1 = jnp.zeros((_L,), jnp.float32)
        for i in range(_PATH):
            w = u_rows[i, pl.ds(0, _L)] * vch[0]
            for j in range(1, _EMB // _L):
                w = w + u_rows[i, pl.ds(j * _L, _L)] * vch[j]
            sv = jnp.broadcast_to(jnp.sum(w), (_L,))
            if i < _L:
                acc0 = jnp.where(lane == i, sv, acc0)
            else:
                acc1 = jnp.where(lane == (i - _L), sv, acc1)

        c0 = codes_v[pl.ds(0, _L)]
        c1 = codes_v[pl.ds(_L, _L)]
        z0 = jnp.where(c0 == 1, acc0, -acc0)
        z1 = jnp.where(c1 == 1, acc1, -acc1)

        p0 = 1.0 / (1.0 + jnp.exp(-z0))
        p1 = 1.0 / (1.0 + jnp.exp(-z1))
        l0 = _log_f32(p0 + 1e-9)
        l1 = _log_f32(p1 + 1e-9)

        lane = lax.iota(jnp.int32, 16)
        l1 = jnp.where(lane < (_PATH - _L), l1, 0.0)
        loss = -jnp.sum(l0 + l1)

        out_v[...] = jnp.broadcast_to(loss, (_L,))
        pltpu.sync_copy(out_v, out_hbm)


@functools.cache
def _build_sc_fn():
  return pl.kernel(
    _sc_body,
    out_type=jax.ShapeDtypeStruct((_L,), jnp.float32),
    mesh=plsc.VectorSubcoreMesh(core_axis_name="c", subcore_axis_name="s"),
    scratch_types=[
        pltpu.VMEM((_ROWS + 16,), jnp.int32),
        pltpu.VMEM((_ROWS,), jnp.int32),
        pltpu.VMEM((_ROWS, _EMB), jnp.float32),
        pltpu.VMEM((8, _EMB), jnp.float32),
        pltpu.VMEM((_L,), jnp.float32),
        pltpu.SemaphoreType.DMA,
        pltpu.SemaphoreType.DMA,
    ],
    compiler_params=pltpu.CompilerParams(needs_layout_passes=False),
  )


@jax.jit
def kernel(in_table, node_table, target_idx, node_ids, codes):
    idx_pad = jnp.zeros((_ROWS,), jnp.int32).at[:_PATH].set(node_ids.astype(jnp.int32))
    tgt_pad = jnp.zeros((8,), jnp.int32).at[0].set(target_idx.astype(jnp.int32))
    codes_pad = jnp.zeros((_ROWS,), jnp.int32).at[:_PATH].set(codes.astype(jnp.int32))
    out = _build_sc_fn()(in_table, node_table, idx_pad, tgt_pad, codes_pad)
    return out[0]


# speedup vs baseline: 35.5831x; 22.3207x over previous
import functools

import jax
import jax.numpy as jnp
from jax import lax
from jax.experimental import pallas as pl
from jax.experimental.pallas import tpu as pltpu
from jax.experimental.pallas import tpu_sc as plsc

_EMB = 64
_PATH = 20
_ROWS = 32
_L = 16
_BLK = 128
_NBUF = 7


def _log_f32(x):
    bits = lax.bitcast_convert_type(x, jnp.int32)
    e = (bits >> 23) - 127
    mbits = (bits & 0x007FFFFF) | 0x3F800000
    m = lax.bitcast_convert_type(mbits, jnp.float32)
    big = m > 1.4142135381698608
    m = jnp.where(big, m * 0.5, m)
    ef = e.astype(jnp.float32) + jnp.where(big, 1.0, 0.0)
    t = (m - 1.0) / (m + 1.0)
    t2 = t * t
    poly = 1.0 + t2 * (
        0.3333333432674408
        + t2 * (0.20000000298023224 + t2 * (0.1428571492433548 + t2 * 0.1111111119389534))
    )
    return ef * 0.6931471805599453 + 2.0 * t * poly


def _extract_col(blk, m_vec):
    return [
        plsc.load_gather(blk, [lax.iota(jnp.int32, 16) + 16 * j, m_vec])
        for j in range(_EMB // _L)
    ]


def _sc_body(in_t, node_t, idx_hbm, tgt_hbm, codes_hbm, out_hbm,
             idx_vtmp, codes_v, v_blk, bufs, out_v, sem_u, sem_v):
    cid = lax.axis_index("c")
    sid = lax.axis_index("s")

    @pl.when(jnp.logical_and(cid == 0, sid == 0))
    def _():
        pltpu.sync_copy(idx_hbm, idx_vtmp.at[pl.ds(0, _ROWS)])
        pltpu.sync_copy(tgt_hbm, idx_vtmp.at[pl.ds(_ROWS, 8)])
        pltpu.sync_copy(codes_hbm, codes_v)

        lane = lax.iota(jnp.int32, 16)
        iv0 = idx_vtmp[pl.ds(0, _L)]
        iv1 = idx_vtmp[pl.ds(_L, _L)]
        idx_scalars = [
            jnp.sum(jnp.where(lane == i, iv0, 0)) for i in range(_L)
        ] + [
            jnp.sum(jnp.where(lane == (i - _L), iv1, 0)) for i in range(_L, _PATH)
        ]
        tgt_scalar = jnp.sum(jnp.where(lane == 0, idx_vtmp[pl.ds(_ROWS, _L)], 0))

        def fetch(tbl, r, dst, sem):
            base = pl.multiple_of((r // _BLK) * _BLK, _BLK)
            cp = pltpu.make_async_copy(tbl.at[:, pl.ds(base, _BLK)], dst, sem)
            cp.start()
            return cp

        cp_v = fetch(in_t, tgt_scalar, v_blk, sem_v)
        waves = [list(range(0, _NBUF)), list(range(_NBUF, 2 * _NBUF)),
                 list(range(2 * _NBUF, _PATH))]
        cps = [fetch(node_t, idx_scalars[i], bufs[k], sem_u)
               for k, i in enumerate(waves[0])]
        cp_v.wait()
        vch = _extract_col(v_blk, jnp.broadcast_to(tgt_scalar & (_BLK - 1), (_L,)))

        acc0 = jnp.zeros((_L,), jnp.float32)
        acc1 = jnp.zeros((_L,), jnp.float32)
        for w, wave in enumerate(waves):
            for cp in cps:
                cp.wait()
            dots = []
            for k, i in enumerate(wave):
                m_vec = jnp.broadcast_to(idx_scalars[i] & (_BLK - 1), (_L,))
                uch = _extract_col(bufs[k], m_vec)
                prod = uch[0] * vch[0]
                for j in range(1, _EMB // _L):
                    prod = prod + uch[j] * vch[j]
                dots.append((i, jnp.sum(prod)))
            if w + 1 < len(waves):
                cps = [fetch(node_t, idx_scalars[i], bufs[k], sem_u)
                       for k, i in enumerate(waves[w + 1])]
            for i, d in dots:
                dv = jnp.broadcast_to(d, (_L,))
                if i < _L:
                    acc0 = jnp.where(lane == i, dv, acc0)
                else:
                    acc1 = jnp.where(lane == (i - _L), dv, acc1)

        c0 = codes_v[pl.ds(0, _L)]
        c1 = codes_v[pl.ds(_L, _L)]
        z0 = jnp.where(c0 == 1, acc0, -acc0)
        z1 = jnp.where(c1 == 1, acc1, -acc1)

        p0 = 1.0 / (1.0 + jnp.exp(-z0))
        p1 = 1.0 / (1.0 + jnp.exp(-z1))
        l0 = _log_f32(p0 + 1e-9)
        l1 = _log_f32(p1 + 1e-9)

        l1 = jnp.where(lane < (_PATH - _L), l1, 0.0)
        loss = -jnp.sum(l0 + l1)

        out_v[...] = jnp.broadcast_to(loss, (_L,))
        pltpu.sync_copy(out_v, out_hbm)


@functools.cache
def _build_sc_fn():
  return pl.kernel(
    _sc_body,
    out_type=jax.ShapeDtypeStruct((_L,), jnp.float32),
    mesh=plsc.VectorSubcoreMesh(core_axis_name="c", subcore_axis_name="s"),
    scratch_types=[
        pltpu.VMEM((_ROWS + 16,), jnp.int32),
        pltpu.VMEM((_ROWS,), jnp.int32),
        pltpu.VMEM((_EMB, _BLK), jnp.float32),
        [pltpu.VMEM((_EMB, _BLK), jnp.float32) for _ in range(_NBUF)],
        pltpu.VMEM((_L,), jnp.float32),
        pltpu.SemaphoreType.DMA,
        pltpu.SemaphoreType.DMA,
    ],
    compiler_params=pltpu.CompilerParams(needs_layout_passes=False),
  )


@jax.jit
def kernel(in_table, node_table, target_idx, node_ids, codes):
    in_t = jnp.swapaxes(in_table, 0, 1)
    node_t = jnp.swapaxes(node_table, 0, 1)
    idx_pad = jnp.zeros((_ROWS,), jnp.int32).at[:_PATH].set(node_ids.astype(jnp.int32))
    tgt_pad = jnp.zeros((8,), jnp.int32).at[0].set(target_idx.astype(jnp.int32))
    codes_pad = jnp.zeros((_ROWS,), jnp.int32).at[:_PATH].set(codes.astype(jnp.int32))
    out = _build_sc_fn()(in_t, node_t, idx_pad, tgt_pad, codes_pad)
    return out[0]


# speedup vs baseline: 38.6113x; 1.0851x over previous
import functools

import jax
import jax.numpy as jnp
from jax import lax
from jax.experimental import pallas as pl
from jax.experimental.pallas import tpu as pltpu
from jax.experimental.pallas import tpu_sc as plsc

_EMB = 64
_PATH = 20
_L = 16
_BLK = 128
_NBUF = 14


def _log_f32(x):
    bits = lax.bitcast_convert_type(x, jnp.int32)
    e = (bits >> 23) - 127
    mbits = (bits & 0x007FFFFF) | 0x3F800000
    m = lax.bitcast_convert_type(mbits, jnp.float32)
    big = m > 1.4142135381698608
    m = jnp.where(big, m * 0.5, m)
    ef = e.astype(jnp.float32) + jnp.where(big, 1.0, 0.0)
    t = (m - 1.0) / (m + 1.0)
    t2 = t * t
    poly = 1.0 + t2 * (
        0.3333333432674408
        + t2 * (0.20000000298023224 + t2 * (0.1428571492433548 + t2 * 0.1111111119389534))
    )
    return ef * 0.6931471805599453 + 2.0 * t * poly


def _extract_col(blk, m_vec):
    return [
        plsc.load_gather(blk, [lax.iota(jnp.int32, 16) + 16 * j, m_vec])
        for j in range(_EMB // _L)
    ]


def _sc_body(in_t, node_t, idx_hbm, tgt_hbm, codes_hbm, out_hbm,
             idx_vtmp, codes_v, v_blk, bufs, out_v, sem_s, sem_u, sem_v):
    cid = lax.axis_index("c")
    sid = lax.axis_index("s")

    @pl.when(jnp.logical_and(cid == 0, sid == 0))
    def _():
        st = [
            pltpu.make_async_copy(idx_hbm, idx_vtmp.at[pl.ds(0, _PATH)], sem_s),
            pltpu.make_async_copy(tgt_hbm, idx_vtmp.at[pl.ds(32, 1)], sem_s),
            pltpu.make_async_copy(codes_hbm, codes_v.at[pl.ds(0, _PATH)], sem_s),
        ]
        for cp in st:
            cp.start()
        for cp in st:
            cp.wait()

        lane = lax.iota(jnp.int32, 16)
        iv0 = idx_vtmp[pl.ds(0, _L)]
        iv1 = idx_vtmp[pl.ds(_L, _L)]
        idx_scalars = [
            jnp.sum(jnp.where(lane == i, iv0, 0)) for i in range(_L)
        ] + [
            jnp.sum(jnp.where(lane == (i - _L), iv1, 0)) for i in range(_L, _PATH)
        ]
        tgt_scalar = jnp.sum(jnp.where(lane == 0, idx_vtmp[pl.ds(32, _L)], 0))

        def fetch(tbl, r, dst, sem):
            base = pl.multiple_of((r // _BLK) * _BLK, _BLK)
            cp = pltpu.make_async_copy(tbl.at[:, pl.ds(base, _BLK)], dst, sem)
            cp.start()
            return cp

        def col(i, k):
            m_vec = jnp.broadcast_to(idx_scalars[i] & (_BLK - 1), (_L,))
            return _extract_col(bufs[k], m_vec)

        def dot(uch, vch):
            prod = uch[0] * vch[0]
            for j in range(1, _EMB // _L):
                prod = prod + uch[j] * vch[j]
            return jnp.sum(prod)

        cp_v = fetch(in_t, tgt_scalar, v_blk, sem_v)
        cps = [fetch(node_t, idx_scalars[i], bufs[i], sem_u)
               for i in range(_NBUF)]
        cp_v.wait()
        vch = _extract_col(v_blk, jnp.broadcast_to(tgt_scalar & (_BLK - 1), (_L,)))
        for cp in cps:
            cp.wait()

        n2 = _PATH - _NBUF
        dots = [(i, dot(col(i, i), vch)) for i in range(n2)]
        cps = [fetch(node_t, idx_scalars[_NBUF + k], bufs[k], sem_u)
               for k in range(n2)]
        dots += [(i, dot(col(i, i), vch)) for i in range(n2, _NBUF)]
        for cp in cps:
            cp.wait()
        dots += [(_NBUF + k, dot(col(_NBUF + k, k), vch)) for k in range(n2)]

        acc0 = jnp.zeros((_L,), jnp.float32)
        acc1 = jnp.zeros((_L,), jnp.float32)
        for i, d in dots:
            dv = jnp.broadcast_to(d, (_L,))
            if i < _L:
                acc0 = jnp.where(lane == i, dv, acc0)
            else:
                acc1 = jnp.where(lane == (i - _L), dv, acc1)

        c0 = codes_v[pl.ds(0, _L)]
        c1 = codes_v[pl.ds(_L, _L)]
        z0 = jnp.where(c0 == 1, acc0, -acc0)
        z1 = jnp.where(c1 == 1, acc1, -acc1)

        p0 = 1.0 / (1.0 + jnp.exp(-z0))
        p1 = 1.0 / (1.0 + jnp.exp(-z1))
        l0 = _log_f32(p0 + 1e-9)
        l1 = _log_f32(p1 + 1e-9)

        l1 = jnp.where(lane < (_PATH - _L), l1, 0.0)
        loss = -jnp.sum(l0 + l1)

        out_v[...] = jnp.broadcast_to(loss, (_L,))
        pltpu.sync_copy(out_v, out_hbm)


@functools.cache
def _build_sc_fn():
  return pl.kernel(
    _sc_body,
    out_type=jax.ShapeDtypeStruct((_L,), jnp.float32),
    mesh=plsc.VectorSubcoreMesh(core_axis_name="c", subcore_axis_name="s"),
    scratch_types=[
        pltpu.VMEM((48,), jnp.int32),
        pltpu.VMEM((32,), jnp.int32),
        pltpu.VMEM((_EMB, _BLK), jnp.float32),
        [pltpu.VMEM((_EMB, _BLK), jnp.float32) for _ in range(_NBUF)],
        pltpu.VMEM((_L,), jnp.float32),
        pltpu.SemaphoreType.DMA,
        pltpu.SemaphoreType.DMA,
        pltpu.SemaphoreType.DMA,
    ],
    compiler_params=pltpu.CompilerParams(needs_layout_passes=False),
  )


@jax.jit
def kernel(in_table, node_table, target_idx, node_ids, codes):
    in_t = jnp.swapaxes(in_table, 0, 1)
    node_t = jnp.swapaxes(node_table, 0, 1)
    out = _build_sc_fn()(
        in_t, node_t,
        node_ids.astype(jnp.int32),
        target_idx.astype(jnp.int32).reshape(1),
        codes.astype(jnp.int32),
    )
    return out[0]


# speedup vs baseline: 46.3014x; 1.1992x over previous
import functools

import jax
import jax.numpy as jnp
from jax import lax
from jax.experimental import pallas as pl
from jax.experimental.pallas import tpu as pltpu
from jax.experimental.pallas import tpu_sc as plsc

_EMB = 64
_PATH = 20
_L = 16
_BLK = 128


def _log_f32(x):
    bits = lax.bitcast_convert_type(x, jnp.int32)
    e = (bits >> 23) - 127
    mbits = (bits & 0x007FFFFF) | 0x3F800000
    m = lax.bitcast_convert_type(mbits, jnp.float32)
    big = m > 1.4142135381698608
    m = jnp.where(big, m * 0.5, m)
    ef = e.astype(jnp.float32) + jnp.where(big, 1.0, 0.0)
    t = (m - 1.0) / (m + 1.0)
    t2 = t * t
    poly = 1.0 + t2 * (
        0.3333333432674408
        + t2 * (0.20000000298023224 + t2 * (0.1428571492433548 + t2 * 0.1111111119389534))
    )
    return ef * 0.6931471805599453 + 2.0 * t * poly


def _extract_col(blk, m_vec):
    return [
        plsc.load_gather(blk, [lax.iota(jnp.int32, 16) + 16 * j, m_vec])
        for j in range(_EMB // _L)
    ]


def _sc_body(in_t, node_t, idx_hbm, tgt_hbm, codes_hbm, out_hbm,
             idx_vtmp, codes_v, v_blk, blk1, blk2, row_v,
             sem_s, sem_u, sem_v):
    cid = lax.axis_index("c")
    sid = lax.axis_index("s")

    @pl.when(cid == 0)
    def _():
        st = [
            pltpu.make_async_copy(idx_hbm, idx_vtmp.at[pl.ds(0, _PATH)], sem_s),
            pltpu.make_async_copy(tgt_hbm, idx_vtmp.at[pl.ds(32, 1)], sem_s),
            pltpu.make_async_copy(codes_hbm, codes_v.at[pl.ds(0, _PATH)], sem_s),
        ]
        for cp in st:
            cp.start()
        for cp in st:
            cp.wait()

        lane = lax.iota(jnp.int32, 16)
        iv0 = idx_vtmp[pl.ds(0, _L)]
        iv1 = idx_vtmp[pl.ds(_L, _L)]
        cv0 = codes_v[pl.ds(0, _L)]
        cv1 = codes_v[pl.ds(_L, _L)]
        mine = lane == sid
        i1 = jnp.sum(jnp.where(mine, iv0, 0))
        has2 = sid < (_PATH - _L)
        i2 = jnp.where(has2, jnp.sum(jnp.where(mine, iv1, 0)), 0)
        cd1 = jnp.sum(jnp.where(mine, cv0, 0))
        cd2 = jnp.sum(jnp.where(mine, cv1, 0))
        tgt = jnp.sum(jnp.where(lane == 0, idx_vtmp[pl.ds(32, _L)], 0))

        def fetch(tbl, r, dst, sem):
            base = pl.multiple_of((r // _BLK) * _BLK, _BLK)
            cp = pltpu.make_async_copy(tbl.at[:, pl.ds(base, _BLK)], dst, sem)
            cp.start()
            return cp

        cps = [fetch(in_t, tgt, v_blk, sem_v),
               fetch(node_t, i1, blk1, sem_u),
               fetch(node_t, i2, blk2, sem_u)]
        for cp in cps:
            cp.wait()

        vch = _extract_col(v_blk, jnp.broadcast_to(tgt & (_BLK - 1), (_L,)))

        def dot(blk, r):
            uch = _extract_col(blk, jnp.broadcast_to(r & (_BLK - 1), (_L,)))
            prod = uch[0] * vch[0]
            for j in range(1, _EMB // _L):
                prod = prod + uch[j] * vch[j]
            return jnp.sum(prod)

        d1 = dot(blk1, i1)
        d2 = dot(blk2, i2)

        zero = jnp.zeros((_L,), jnp.float32)
        dvec = jnp.where(lane == 0, jnp.broadcast_to(d1, (_L,)),
                         jnp.where(lane == 1, jnp.broadcast_to(d2, (_L,)), zero))
        cdvec = jnp.where(lane == 0, jnp.broadcast_to(cd1, (_L,)),
                          jnp.broadcast_to(cd2, (_L,)))
        z = jnp.where(cdvec == 1, dvec, -dvec)

        p = 1.0 / (1.0 + jnp.exp(-z))
        l = _log_f32(p + 1e-9)
        valid = jnp.logical_or(lane == 0, jnp.logical_and(lane == 1, has2))
        row_v[...] = jnp.where(valid, -l, 0.0)
        pltpu.sync_copy(row_v, out_hbm.at[sid])


@functools.cache
def _build_sc_fn():
  return pl.kernel(
    _sc_body,
    out_type=jax.ShapeDtypeStruct((_L, _L), jnp.float32),
    mesh=plsc.VectorSubcoreMesh(core_axis_name="c", subcore_axis_name="s"),
    scratch_types=[
        pltpu.VMEM((48,), jnp.int32),
        pltpu.VMEM((32,), jnp.int32),
        pltpu.VMEM((_EMB, _BLK), jnp.float32),
        pltpu.VMEM((_EMB, _BLK), jnp.float32),
        pltpu.VMEM((_EMB, _BLK), jnp.float32),
        pltpu.VMEM((_L,), jnp.float32),
        pltpu.SemaphoreType.DMA,
        pltpu.SemaphoreType.DMA,
        pltpu.SemaphoreType.DMA,
    ],
    compiler_params=pltpu.CompilerParams(needs_layout_passes=False),
  )


@jax.jit
def kernel(in_table, node_table, target_idx, node_ids, codes):
    in_t = jnp.swapaxes(in_table, 0, 1)
    node_t = jnp.swapaxes(node_table, 0, 1)
    out = _build_sc_fn()(
        in_t, node_t,
        node_ids.astype(jnp.int32),
        target_idx.astype(jnp.int32).reshape(1),
        codes.astype(jnp.int32),
    )
    return jnp.sum(out)


# speedup vs baseline: 49.8460x; 1.0766x over previous
import functools

import jax
import jax.numpy as jnp
from jax import lax
from jax.experimental import pallas as pl
from jax.experimental.pallas import tpu as pltpu
from jax.experimental.pallas import tpu_sc as plsc

_EMB = 64
_PATH = 20
_L = 16
_BLK = 128


def _log_f32(x):
    bits = lax.bitcast_convert_type(x, jnp.int32)
    e = (bits >> 23) - 127
    mbits = (bits & 0x007FFFFF) | 0x3F800000
    m = lax.bitcast_convert_type(mbits, jnp.float32)
    big = m > 1.4142135381698608
    m = jnp.where(big, m * 0.5, m)
    ef = e.astype(jnp.float32) + jnp.where(big, 1.0, 0.0)
    t = (m - 1.0) / (m + 1.0)
    t2 = t * t
    poly = 1.0 + t2 * (
        0.3333333432674408
        + t2 * (0.20000000298023224 + t2 * (0.1428571492433548 + t2 * 0.1111111119389534))
    )
    return ef * 0.6931471805599453 + 2.0 * t * poly


def _extract_col(blk, m_vec):
    return [
        plsc.load_gather(blk, [lax.iota(jnp.int32, 16) + 16 * j, m_vec])
        for j in range(_EMB // _L)
    ]


def _sc_body(in_t, node_t, idx_hbm, tgt_hbm, codes_hbm, out_hbm,
             idx_vtmp, codes_v, v_blk, blk1, blk2, row_v,
             sem_s, sem_u, sem_v):
    cid = lax.axis_index("c")
    sid = lax.axis_index("s")

    @pl.when(cid == 0)
    def _():
        st = [
            pltpu.make_async_copy(idx_hbm, idx_vtmp.at[pl.ds(0, _PATH)], sem_s),
            pltpu.make_async_copy(tgt_hbm, idx_vtmp.at[pl.ds(32, 1)], sem_s),
            pltpu.make_async_copy(codes_hbm, codes_v.at[pl.ds(0, _PATH)], sem_s),
        ]
        for cp in st:
            cp.start()
        for cp in st:
            cp.wait()

        lane = lax.iota(jnp.int32, 16)
        iv0 = idx_vtmp[pl.ds(0, _L)]
        iv1 = idx_vtmp[pl.ds(_L, _L)]
        cv0 = codes_v[pl.ds(0, _L)]
        cv1 = codes_v[pl.ds(_L, _L)]
        mine = lane == sid
        i1 = jnp.sum(jnp.where(mine, iv0, 0))
        has2 = sid < (_PATH - _L)
        i2 = jnp.where(has2, jnp.sum(jnp.where(mine, iv1, 0)), 0)
        cd1 = jnp.sum(jnp.where(mine, cv0, 0))
        cd2 = jnp.sum(jnp.where(mine, cv1, 0))
        tgt = jnp.sum(jnp.where(lane == 0, idx_vtmp[pl.ds(32, _L)], 0))

        def fetch(tbl, r, dst, sem):
            base = pl.multiple_of((r // _BLK) * _BLK, _BLK)
            cp = pltpu.make_async_copy(tbl.at[:, pl.ds(base, _BLK)], dst, sem)
            cp.start()
            return cp

        cps = [fetch(in_t, tgt, v_blk, sem_v),
               fetch(node_t, i1, blk1, sem_u),
               fetch(node_t, i2, blk2, sem_u)]
        for cp in cps:
            cp.wait()

        vch = _extract_col(v_blk, jnp.broadcast_to(tgt & (_BLK - 1), (_L,)))

        def dot(blk, r):
            uch = _extract_col(blk, jnp.broadcast_to(r & (_BLK - 1), (_L,)))
            prod = uch[0] * vch[0]
            for j in range(1, _EMB // _L):
                prod = prod + uch[j] * vch[j]
            return jnp.sum(prod)

        d1 = dot(blk1, i1)
        d2 = dot(blk2, i2)

        zero = jnp.zeros((_L,), jnp.float32)
        dvec = jnp.where(lane == 0, jnp.broadcast_to(d1, (_L,)),
                         jnp.where(lane == 1, jnp.broadcast_to(d2, (_L,)), zero))
        cdvec = jnp.where(lane == 0, jnp.broadcast_to(cd1, (_L,)),
                          jnp.broadcast_to(cd2, (_L,)))
        z = jnp.where(cdvec == 1, dvec, -dvec)

        p = 1.0 / (1.0 + jnp.exp(-z))
        l = _log_f32(p + 1e-9)
        valid = jnp.logical_or(lane == 0, jnp.logical_and(lane == 1, has2))
        row_v[...] = jnp.where(valid, -l, 0.0)
        pltpu.sync_copy(row_v, out_hbm.at[sid])


@functools.cache
def _build_sc_fn():
  return pl.kernel(
    _sc_body,
    out_type=jax.ShapeDtypeStruct((_L, _L), jnp.float32),
    mesh=plsc.VectorSubcoreMesh(
        core_axis_name="c", subcore_axis_name="s", num_cores=1
    ),
    scratch_types=[
        pltpu.VMEM((48,), jnp.int32),
        pltpu.VMEM((32,), jnp.int32),
        pltpu.VMEM((_EMB, _BLK), jnp.float32),
        pltpu.VMEM((_EMB, _BLK), jnp.float32),
        pltpu.VMEM((_EMB, _BLK), jnp.float32),
        pltpu.VMEM((_L,), jnp.float32),
        pltpu.SemaphoreType.DMA,
        pltpu.SemaphoreType.DMA,
        pltpu.SemaphoreType.DMA,
    ],
    compiler_params=pltpu.CompilerParams(needs_layout_passes=False),
  )


@jax.jit
def kernel(in_table, node_table, target_idx, node_ids, codes):
    in_t = jnp.swapaxes(in_table, 0, 1)
    node_t = jnp.swapaxes(node_table, 0, 1)
    out = _build_sc_fn()(
        in_t, node_t,
        node_ids.astype(jnp.int32),
        target_idx.astype(jnp.int32).reshape(1),
        codes.astype(jnp.int32),
    )
    return jnp.sum(out)


# speedup vs baseline: 50.0037x; 1.0032x over previous
import functools

import jax
import jax.numpy as jnp
from jax import lax
from jax.experimental import pallas as pl
from jax.experimental.pallas import tpu as pltpu
from jax.experimental.pallas import tpu_sc as plsc

_EMB = 64
_PATH = 20
_L = 16
_BLK = 128


def _log_f32(x):
    bits = lax.bitcast_convert_type(x, jnp.int32)
    e = (bits >> 23) - 127
    mbits = (bits & 0x007FFFFF) | 0x3F800000
    m = lax.bitcast_convert_type(mbits, jnp.float32)
    big = m > 1.4142135381698608
    m = jnp.where(big, m * 0.5, m)
    ef = e.astype(jnp.float32) + jnp.where(big, 1.0, 0.0)
    t = (m - 1.0) / (m + 1.0)
    t2 = t * t
    poly = 1.0 + t2 * (
        0.3333333432674408
        + t2 * (0.20000000298023224 + t2 * (0.1428571492433548 + t2 * 0.1111111119389534))
    )
    return ef * 0.6931471805599453 + 2.0 * t * poly


def _extract_col(blk, m_vec):
    return [
        plsc.load_gather(blk, [lax.iota(jnp.int32, 16) + 16 * j, m_vec])
        for j in range(_EMB // _L)
    ]


def _sc_body(in_t, node_t, idx_hbm, tgt_hbm, codes_hbm, out_hbm,
             idx_vtmp, codes_v, v_blk, blk1, blk2, row_v,
             sem_s, sem_u, sem_v):
    cid = lax.axis_index("c")
    sid = lax.axis_index("s")

    @pl.when(cid == 0)
    def _():
        st = [
            pltpu.make_async_copy(idx_hbm, idx_vtmp.at[pl.ds(0, _PATH)], sem_s),
            pltpu.make_async_copy(tgt_hbm, idx_vtmp.at[pl.ds(32, 1)], sem_s),
            pltpu.make_async_copy(codes_hbm, codes_v.at[pl.ds(0, _PATH)], sem_s),
        ]
        for cp in st:
            cp.start()
        for cp in st:
            cp.wait()

        lane = lax.iota(jnp.int32, 16)
        iv0 = idx_vtmp[pl.ds(0, _L)]
        iv1 = idx_vtmp[pl.ds(_L, _L)]
        cv0 = codes_v[pl.ds(0, _L)]
        cv1 = codes_v[pl.ds(_L, _L)]
        mine = lane == sid
        i1 = jnp.sum(jnp.where(mine, iv0, 0))
        has2 = sid < (_PATH - _L)
        i2 = jnp.where(has2, jnp.sum(jnp.where(mine, iv1, 0)), 0)
        cd1 = jnp.sum(jnp.where(mine, cv0, 0))
        cd2 = jnp.sum(jnp.where(mine, cv1, 0))
        tgt = jnp.sum(jnp.where(lane == 0, idx_vtmp[pl.ds(32, _L)], 0))

        def fetch(tbl, r, dst, sem):
            base = pl.multiple_of((r // _BLK) * _BLK, _BLK)
            cp = pltpu.make_async_copy(tbl.at[:, pl.ds(base, _BLK)], dst, sem)
            cp.start()
            return cp

        cps = [fetch(in_t, tgt, v_blk, sem_v),
               fetch(node_t, i1, blk1, sem_u)]

        @pl.when(has2)
        def _():
            fetch(node_t, i2, blk2, sem_u).wait()

        for cp in cps:
            cp.wait()

        vch = _extract_col(v_blk, jnp.broadcast_to(tgt & (_BLK - 1), (_L,)))

        def dot(blk, r):
            uch = _extract_col(blk, jnp.broadcast_to(r & (_BLK - 1), (_L,)))
            prod = uch[0] * vch[0]
            for j in range(1, _EMB // _L):
                prod = prod + uch[j] * vch[j]
            return jnp.sum(prod)

        d1 = dot(blk1, i1)
        d2 = dot(blk2, i2)

        zero = jnp.zeros((_L,), jnp.float32)
        dvec = jnp.where(lane == 0, jnp.broadcast_to(d1, (_L,)),
                         jnp.where(lane == 1, jnp.broadcast_to(d2, (_L,)), zero))
        cdvec = jnp.where(lane == 0, jnp.broadcast_to(cd1, (_L,)),
                          jnp.broadcast_to(cd2, (_L,)))
        z = jnp.where(cdvec == 1, dvec, -dvec)

        p = 1.0 / (1.0 + jnp.exp(-z))
        l = _log_f32(p + 1e-9)
        valid = jnp.logical_or(lane == 0, jnp.logical_and(lane == 1, has2))
        row_v[...] = jnp.where(valid, -l, 0.0)
        pltpu.sync_copy(row_v, out_hbm.at[sid])


@functools.cache
def _build_sc_fn():
  return pl.kernel(
    _sc_body,
    out_type=jax.ShapeDtypeStruct((_L, _L), jnp.float32),
    mesh=plsc.VectorSubcoreMesh(
        core_axis_name="c", subcore_axis_name="s", num_cores=1
    ),
    scratch_types=[
        pltpu.VMEM((48,), jnp.int32),
        pltpu.VMEM((32,), jnp.int32),
        pltpu.VMEM((_EMB, _BLK), jnp.float32),
        pltpu.VMEM((_EMB, _BLK), jnp.float32),
        pltpu.VMEM((_EMB, _BLK), jnp.float32),
        pltpu.VMEM((_L,), jnp.float32),
        pltpu.SemaphoreType.DMA,
        pltpu.SemaphoreType.DMA,
        pltpu.SemaphoreType.DMA,
    ],
    compiler_params=pltpu.CompilerParams(needs_layout_passes=False),
  )


@jax.jit
def kernel(in_table, node_table, target_idx, node_ids, codes):
    in_t = jnp.swapaxes(in_table, 0, 1)
    node_t = jnp.swapaxes(node_table, 0, 1)
    out = _build_sc_fn()(
        in_t, node_t,
        node_ids.astype(jnp.int32),
        target_idx.astype(jnp.int32).reshape(1),
        codes.astype(jnp.int32),
    )
    return jnp.sum(out)
